# baseline (speedup 1.0000x reference)
import jax
import jax.numpy as jnp
from jax.experimental import pallas as pl
from jax.experimental.pallas import tpu as pltpu

NR_AGENTS = 16
MAX_NB = 16
NB = MAX_NB - 1
UAV_OBS = 130
FEAT_DIM = UAV_OBS - 2
ME_DIM = 256
ME_DIM_SINGLE = NB * UAV_OBS
VFPI_ADD = UAV_OBS + 2 + 64 * 4
FEAT_ALL = ME_DIM_SINGLE + VFPI_ADD
BATCH = 1024
AP = 2
BB = AP * BATCH
CHUNK = 3


def _fused(n_ref, w_ref,
           f_ref,
           w0_ref, b0_ref, w1_ref, b1_ref,
           wpv0a_ref, wpv0b_ref, bpv0_ref,
           wp1_ref, bp1_ref, wv1_ref, bv1_ref,
           pi_ref, vf_ref,
           acc_ref):
    g = pl.program_id(0)

    acc_ref[...] = jnp.zeros_like(acc_ref)
    for half in range(AP):
        a = g * AP + half
        n_i = n_ref[a]
        r0 = half * BATCH
        for c in range(0, NB, CHUNK):
            @pl.when(c < n_i)
            def _():
                acc = acc_ref[r0:r0 + BATCH, :]
                for j in range(c, min(c + CHUNK, NB)):
                    x = f_ref[r0:r0 + BATCH, UAV_OBS * j:UAV_OBS * j + FEAT_DIM]
                    h = jnp.tanh(
                        jnp.dot(x, w0_ref[...],
                                preferred_element_type=jnp.float32)
                        + b0_ref[...])
                    s = jnp.tanh(
                        jnp.dot(h, w1_ref[...],
                                preferred_element_type=jnp.float32)
                        + b1_ref[...])
                    acc = acc + s * w_ref[a, j]
                acc_ref[r0:r0 + BATCH, :] = acc

    lat = acc_ref[...]
    selfx = f_ref[:, ME_DIM_SINGLE:FEAT_ALL]

    t = jnp.tanh(
        jnp.dot(lat, wpv0a_ref[...], preferred_element_type=jnp.float32)
        + jnp.dot(selfx, wpv0b_ref[...], preferred_element_type=jnp.float32)
        + bpv0_ref[...])
    pi_ref[...] = jnp.tanh(
        jnp.dot(t[:, :ME_DIM], wp1_ref[...],
                preferred_element_type=jnp.float32) + bp1_ref[...])
    vf_ref[...] = jnp.tanh(
        jnp.dot(t[:, ME_DIM:], wv1_ref[...],
                preferred_element_type=jnp.float32) + bv1_ref[...])


@jax.jit
def kernel(features, Ws0, bs0, Ws1, bs1, Wp0, bp0, Wp1, bp1, Wv0, bv0, Wv1, bv1):
    head = features[:NR_AGENTS, :ME_DIM_SINGLE].reshape(NR_AGENTS, NB, UAV_OBS)
    n = jnp.floor(jnp.sum(head[:, :, FEAT_DIM], axis=1))
    n_int = n.astype(jnp.int32)
    scale = jnp.where(n < 1.0, 0.0, 1.0 / jnp.maximum(n, 1.0))
    wtab = jnp.where(jnp.arange(NB, dtype=jnp.float32)[None, :] < n[:, None],
                     scale[:, None], 0.0)

    Wpv0 = jnp.concatenate([Wp0, Wv0], axis=1)
    bpv0 = jnp.concatenate([bp0, bv0])

    row2 = lambda b: b.reshape(1, -1)
    grid = (NR_AGENTS // AP,)

    const = lambda *shape: pl.BlockSpec(shape, lambda g, *_: (0,) * len(shape))
    out_shape = jax.ShapeDtypeStruct((NR_AGENTS * BATCH, ME_DIM), jnp.float32)
    out_spec = pl.BlockSpec((BB, ME_DIM), lambda g, *_: (g, 0))

    pi, vf = pl.pallas_call(
        _fused,
        grid_spec=pltpu.PrefetchScalarGridSpec(
            num_scalar_prefetch=2,
            grid=grid,
            in_specs=[
                pl.BlockSpec((BB, FEAT_ALL), lambda g, *_: (g, 0)),
                const(FEAT_DIM, 256), const(1, 256),
                const(256, 256), const(1, 256),
                const(ME_DIM, 2 * ME_DIM), const(VFPI_ADD, 2 * ME_DIM),
                const(1, 2 * ME_DIM),
                const(256, 256), const(1, 256),
                const(256, 256), const(1, 256),
            ],
            out_specs=[out_spec, out_spec],
            scratch_shapes=[pltpu.VMEM((BB, ME_DIM), jnp.float32)],
        ),
        out_shape=[out_shape, out_shape],
        compiler_params=pltpu.CompilerParams(
            dimension_semantics=("arbitrary",),
        ),
    )(n_int, wtab,
      features,
      Ws0, row2(bs0), Ws1, row2(bs1),
      Wpv0[:ME_DIM], Wpv0[ME_DIM:], row2(bpv0),
      Wp1, row2(bp1), Wv1, row2(bv1))

    pi = pi.reshape(NR_AGENTS, BATCH, ME_DIM)
    vf = vf.reshape(NR_AGENTS, BATCH, ME_DIM)
    return (pi, vf)


# speedup vs baseline: 1.0058x; 1.0058x over previous
import jax
import jax.numpy as jnp
from jax.experimental import pallas as pl
from jax.experimental.pallas import tpu as pltpu

NR_AGENTS = 16
MAX_NB = 16
NB = MAX_NB - 1
UAV_OBS = 130
FEAT_DIM = UAV_OBS - 2
ME_DIM = 256
ME_DIM_SINGLE = NB * UAV_OBS
VFPI_ADD = UAV_OBS + 2 + 64 * 4
FEAT_ALL = ME_DIM_SINGLE + VFPI_ADD
BATCH = 1024
BB = 1024
NBB = BATCH // BB
CHUNK = 5


def _fused(n_ref, w_ref,
           f_ref,
           w0_ref, b0_ref, w1_ref, b1_ref,
           wpv0a_ref, wpv0b_ref, bpv0_ref,
           wp1_ref, bp1_ref, wv1_ref, bv1_ref,
           pi_ref, vf_ref,
           acc_ref):
    a = pl.program_id(0)
    n_i = n_ref[a]

    acc_ref[...] = jnp.zeros_like(acc_ref)
    for c in range(0, NB, CHUNK):
        @pl.when(c < n_i)
        def _():
            acc = acc_ref[...]
            for j in range(c, min(c + CHUNK, NB)):
                x = f_ref[:, UAV_OBS * j:UAV_OBS * j + FEAT_DIM]
                h = jnp.tanh(
                    jnp.dot(x, w0_ref[...], preferred_element_type=jnp.float32)
                    + b0_ref[...])
                s = jnp.tanh(
                    jnp.dot(h, w1_ref[...], preferred_element_type=jnp.float32)
                    + b1_ref[...])
                acc = acc + s * w_ref[a, j]
            acc_ref[...] = acc

    lat = acc_ref[...]
    selfx = f_ref[:, ME_DIM_SINGLE:FEAT_ALL]

    t = jnp.tanh(
        jnp.dot(lat, wpv0a_ref[...], preferred_element_type=jnp.float32)
        + jnp.dot(selfx, wpv0b_ref[...], preferred_element_type=jnp.float32)
        + bpv0_ref[...])
    pi_ref[...] = jnp.tanh(
        jnp.dot(t[:, :ME_DIM], wp1_ref[...],
                preferred_element_type=jnp.float32) + bp1_ref[...])
    vf_ref[...] = jnp.tanh(
        jnp.dot(t[:, ME_DIM:], wv1_ref[...],
                preferred_element_type=jnp.float32) + bv1_ref[...])


@jax.jit
def kernel(features, Ws0, bs0, Ws1, bs1, Wp0, bp0, Wp1, bp1, Wv0, bv0, Wv1, bv1):
    head = features[:NR_AGENTS, :ME_DIM_SINGLE].reshape(NR_AGENTS, NB, UAV_OBS)
    n = jnp.floor(jnp.sum(head[:, :, FEAT_DIM], axis=1))
    n_int = n.astype(jnp.int32)
    scale = jnp.where(n < 1.0, 0.0, 1.0 / jnp.maximum(n, 1.0))
    wtab = jnp.where(jnp.arange(NB, dtype=jnp.float32)[None, :] < n[:, None],
                     scale[:, None], 0.0)

    Wpv0 = jnp.concatenate([Wp0, Wv0], axis=1)
    bpv0 = jnp.concatenate([bp0, bv0])

    row2 = lambda b: b.reshape(1, -1)
    grid = (NR_AGENTS, NBB)

    const = lambda *shape: pl.BlockSpec(shape, lambda a, bb, *_: (0,) * len(shape))
    out_shape = jax.ShapeDtypeStruct((NR_AGENTS * BATCH, ME_DIM), jnp.float32)
    out_spec = pl.BlockSpec((BB, ME_DIM), lambda a, bb, *_: (a * NBB + bb, 0))

    pi, vf = pl.pallas_call(
        _fused,
        grid_spec=pltpu.PrefetchScalarGridSpec(
            num_scalar_prefetch=2,
            grid=grid,
            in_specs=[
                pl.BlockSpec((BB, FEAT_ALL), lambda a, bb, *_: (a * NBB + bb, 0)),
                const(FEAT_DIM, 256), const(1, 256),
                const(256, 256), const(1, 256),
                const(ME_DIM, 2 * ME_DIM), const(VFPI_ADD, 2 * ME_DIM),
                const(1, 2 * ME_DIM),
                const(256, 256), const(1, 256),
                const(256, 256), const(1, 256),
            ],
            out_specs=[out_spec, out_spec],
            scratch_shapes=[pltpu.VMEM((BB, ME_DIM), jnp.float32)],
        ),
        out_shape=[out_shape, out_shape],
        compiler_params=pltpu.CompilerParams(
            dimension_semantics=("arbitrary", "arbitrary"),
        ),
    )(n_int, wtab,
      features,
      Ws0, row2(bs0), Ws1, row2(bs1),
      Wpv0[:ME_DIM], Wpv0[ME_DIM:], row2(bpv0),
      Wp1, row2(bp1), Wv1, row2(bv1))

    pi = pi.reshape(NR_AGENTS, BATCH, ME_DIM)
    vf = vf.reshape(NR_AGENTS, BATCH, ME_DIM)
    return (pi, vf)
